# baseline (speedup 1.0000x reference)
import functools

import jax
import jax.numpy as jnp
from jax import lax
from jax.experimental import pallas as pl
from jax.experimental.pallas import tpu as pltpu
from jax.experimental.pallas import tpu_sc as plsc

N, E, D, D_EDGE = 10000, 320000, 128, 16
LANES = 16

NC, NS = 2, 16
NW = NC * NS
EPW = E // NW
CHUNK = 40
NCHUNK = EPW // CHUNK
N_PAD = 10240
ROWS_PER_TILE = N_PAD // NS
WB = 128

A_GRID = 40
BLK_E = E // A_GRID
BLK_N = 256


def _stage_a_body(x_ref, eattr_ref, wm_ref, we_ref, bm_ref, xm_ref, ea_ref):
    xm_ref[...] = jnp.dot(x_ref[...], wm_ref[...],
                          preferred_element_type=jnp.float32)
    ea_ref[...] = jnp.dot(eattr_ref[...], we_ref[...],
                          preferred_element_type=jnp.float32) + bm_ref[...]


_stage_a = pl.pallas_call(
    _stage_a_body,
    grid=(A_GRID,),
    in_specs=[
        pl.BlockSpec((BLK_N, D), lambda i: (i, 0)),
        pl.BlockSpec((BLK_E, D_EDGE), lambda i: (i, 0)),
        pl.BlockSpec((D, D), lambda i: (0, 0)),
        pl.BlockSpec((D_EDGE, D), lambda i: (0, 0)),
        pl.BlockSpec((1, D), lambda i: (0, 0)),
    ],
    out_specs=[
        pl.BlockSpec((BLK_N, D), lambda i: (i, 0)),
        pl.BlockSpec((BLK_E, D), lambda i: (i, 0)),
    ],
    out_shape=[
        jax.ShapeDtypeStruct((N, D), jnp.float32),
        jax.ShapeDtypeStruct((E, D), jnp.float32),
    ],
)


def _stage_b_build():
    mesh = plsc.VectorSubcoreMesh(core_axis_name="c", subcore_axis_name="s")

    @functools.partial(
        pl.kernel,
        mesh=mesh,
        out_type=jax.ShapeDtypeStruct((NC, N_PAD, D), jnp.float32),
        scratch_types=[
            pltpu.VMEM((2, CHUNK), jnp.int32),
            pltpu.VMEM((2, CHUNK), jnp.int32),
            pltpu.VMEM((2, CHUNK, D), jnp.float32),
            pltpu.VMEM((2, CHUNK, D), jnp.float32),
            pltpu.VMEM_SHARED((N_PAD, D), jnp.float32),
            pltpu.SemaphoreType.DMA,
            pltpu.SemaphoreType.DMA,
            pltpu.SemaphoreType.DMA,
            pltpu.SemaphoreType.DMA,
            pltpu.SemaphoreType.DMA,
        ],
    )
    def stage_b(xm_hbm, ea_hbm, zeros_hbm, src_hbm, dst_hbm, out_hbm,
                src_v, dst_v, rows_v, ea_v, agg_sh,
                sem_g, sem_e, sem_d, sem_s, sem_sc):
        c = lax.axis_index("c")
        s = lax.axis_index("s")
        wid = c * NS + s

        def gather_d(b):
            return pltpu.make_async_copy(xm_hbm.at[src_v.at[b]],
                                         rows_v.at[b], sem_g)

        def src_d(ci, b):
            return pltpu.make_async_copy(src_hbm.at[wid, ci], src_v.at[b],
                                         sem_s)

        def ea_d(ci, b):
            base = wid * EPW + ci * CHUNK
            return pltpu.make_async_copy(ea_hbm.at[pl.ds(base, CHUNK)],
                                         ea_v.at[b], sem_e)

        def dst_d(ci, b):
            return pltpu.make_async_copy(dst_hbm.at[wid, ci], dst_v.at[b],
                                         sem_d)

        def scatter_d(b):
            return pltpu.make_async_copy(ea_v.at[b],
                                         agg_sh.at[dst_v.at[b]], sem_sc)

        pltpu.sync_copy(zeros_hbm.at[pl.ds(s * ROWS_PER_TILE, ROWS_PER_TILE)],
                        agg_sh.at[pl.ds(s * ROWS_PER_TILE, ROWS_PER_TILE)])
        plsc.subcore_barrier()
        src_d(0, 0).start()
        src_d(1, 1).start()
        src_d(0, 0).wait()
        src_d(1, 1).wait()
        gather_d(0).start()
        gather_d(1).start()
        ea_d(0, 0).start()
        dst_d(0, 0).start()

        def outer_body(g, carry):
            for b in range(2):
                ci = 2 * g + b
                o = 1 - b

                @pl.when(ci >= 1)
                def _():
                    scatter_d(o).wait()

                @pl.when(ci + 1 < NCHUNK)
                def _():
                    ea_d(ci + 1, o).start()
                    dst_d(ci + 1, o).start()

                gather_d(b).wait()

                @pl.when(ci + 2 < NCHUNK)
                def _():
                    src_d(ci + 2, b).start()

                ea_d(ci, b).wait()
                dst_d(ci, b).wait()

                @plsc.parallel_loop(0, CHUNK, unroll=4)
                def row_body(i):
                    for j in range(D // LANES):
                        sl = pl.ds(j * LANES, LANES)
                        v = rows_v[b, i, sl] + ea_v[b, i, sl]
                        ea_v[b, i, sl] = jnp.maximum(v, 0.0)
                scatter_d(b).start(add=True)

                @pl.when(ci + 2 < NCHUNK)
                def _():
                    src_d(ci + 2, b).wait()
                    gather_d(b).start()
            return carry

        lax.fori_loop(0, NCHUNK // 2, outer_body, 0)
        scatter_d(1).wait()
        plsc.subcore_barrier()

        for b in range(ROWS_PER_TILE // CHUNK):
            r0 = s * ROWS_PER_TILE + b * CHUNK
            pltpu.sync_copy(agg_sh.at[pl.ds(r0, CHUNK)], ea_v.at[0])
            pltpu.sync_copy(ea_v.at[0], out_hbm.at[c, pl.ds(r0, CHUNK)])

    return stage_b


_stage_b = _stage_b_build()


def _stage_c_body(agg_ref, x_ref, wa_ref, ws_ref, bo_ref, a_ref, g_ref,
                  be_ref, out_ref):
    agg = (agg_ref[0] + agg_ref[1])[:N]
    h = jnp.dot(agg, wa_ref[...], preferred_element_type=jnp.float32)
    h = h + jnp.dot(x_ref[...], ws_ref[...], preferred_element_type=jnp.float32)
    h = h + bo_ref[...]
    slope = a_ref[0, 0]
    h = jnp.where(h >= 0, h, slope * h)
    mean = jnp.mean(h, axis=0, keepdims=True)
    var = jnp.mean((h - mean) ** 2, axis=0, keepdims=True)
    out_ref[...] = (h - mean) / jnp.sqrt(var + 1e-5) * g_ref[...] + be_ref[...]


_stage_c = pl.pallas_call(
    _stage_c_body,
    out_shape=jax.ShapeDtypeStruct((N, D), jnp.float32),
)


def kernel(x, edge_index, edge_attr, W_msg, W_edge, b_msg, W_agg, W_self,
           b_out, a, gamma, beta):
    src = edge_index[0].reshape(NW, NCHUNK, CHUNK)
    dst = edge_index[1].reshape(NW, NCHUNK, CHUNK)
    xm, ea = _stage_a(x, edge_attr, W_msg, W_edge, b_msg.reshape(1, D))
    zeros = jnp.zeros((N_PAD, D), dtype=jnp.float32)
    agg = _stage_b(xm, ea, zeros, src, dst)
    out = _stage_c(agg, x, W_agg, W_self, b_out.reshape(1, D),
                   a.reshape(1, 1), gamma.reshape(1, D), beta.reshape(1, D))
    return out


# speedup vs baseline: 1.0766x; 1.0766x over previous
import functools

import jax
import jax.numpy as jnp
from jax import lax
from jax.experimental import pallas as pl
from jax.experimental.pallas import tpu as pltpu
from jax.experimental.pallas import tpu_sc as plsc

N, E, D, D_EDGE = 10000, 320000, 128, 16
LANES = 16

_PI = [32 * j + (2 * k < 32) * 0 + (k // 2 if k % 2 == 0 else 16 + k // 2)
       for j in range(D // 32) for k in range(32)]

NC, NS = 2, 16
NW = NC * NS
EPW = E // NW
CHUNK = 80
NCHUNK = EPW // CHUNK
N_PAD = 10240
ROWS_PER_TILE = N_PAD // NS
WB = 128

A_GRID = 40
BLK_E = E // A_GRID
BLK_N = 256


def _stage_a_body(x_ref, eattr_ref, wm_ref, we_ref, bm_ref, xm_ref, ea_ref):
    xm_ref[...] = jnp.dot(x_ref[...], wm_ref[...],
                          preferred_element_type=jnp.float32)
    ea_ref[...] = (jnp.dot(eattr_ref[...], we_ref[...],
                           preferred_element_type=jnp.float32)
                   + bm_ref[...]).astype(jnp.bfloat16)


_stage_a = pl.pallas_call(
    _stage_a_body,
    grid=(A_GRID,),
    in_specs=[
        pl.BlockSpec((BLK_N, D), lambda i: (i, 0)),
        pl.BlockSpec((BLK_E, D_EDGE), lambda i: (i, 0)),
        pl.BlockSpec((D, D), lambda i: (0, 0)),
        pl.BlockSpec((D_EDGE, D), lambda i: (0, 0)),
        pl.BlockSpec((1, D), lambda i: (0, 0)),
    ],
    out_specs=[
        pl.BlockSpec((BLK_N, D), lambda i: (i, 0)),
        pl.BlockSpec((BLK_E, D), lambda i: (i, 0)),
    ],
    out_shape=[
        jax.ShapeDtypeStruct((N, D), jnp.float32),
        jax.ShapeDtypeStruct((E, D), jnp.bfloat16),
    ],
)


def _stage_b_build():
    mesh = plsc.VectorSubcoreMesh(core_axis_name="c", subcore_axis_name="s")

    @functools.partial(
        pl.kernel,
        mesh=mesh,
        out_type=jax.ShapeDtypeStruct((NC, N_PAD, D), jnp.float32),
        scratch_types=[
            pltpu.VMEM((2, CHUNK), jnp.int32),
            pltpu.VMEM((2, CHUNK), jnp.int32),
            pltpu.VMEM((2, CHUNK, D), jnp.float32),
            pltpu.VMEM((2, CHUNK, D // 2), jnp.uint32),
            pltpu.VMEM_SHARED((N_PAD, D), jnp.float32),
            pltpu.SemaphoreType.DMA,
            pltpu.SemaphoreType.DMA,
            pltpu.SemaphoreType.DMA,
            pltpu.SemaphoreType.DMA,
            pltpu.SemaphoreType.DMA,
        ],
    )
    def stage_b(xm_hbm, ea_hbm, src_hbm, dst_hbm, out_hbm,
                src_v, dst_v, rows_v, ea_v, agg_sh,
                sem_g, sem_e, sem_d, sem_s, sem_sc):
        c = lax.axis_index("c")
        s = lax.axis_index("s")
        wid = c * NS + s

        def gather_d(b):
            return pltpu.make_async_copy(xm_hbm.at[src_v.at[b]],
                                         rows_v.at[b], sem_g)

        def src_d(ci, b):
            base = wid * EPW + ci * CHUNK
            return pltpu.make_async_copy(src_hbm.at[pl.ds(base, CHUNK)],
                                         src_v.at[b], sem_s)

        def ea_d(ci, b):
            base = wid * EPW + ci * CHUNK
            return pltpu.make_async_copy(ea_hbm.at[pl.ds(base, CHUNK)],
                                         ea_v.at[b], sem_e)

        def dst_d(ci, b):
            base = wid * EPW + ci * CHUNK
            return pltpu.make_async_copy(dst_hbm.at[pl.ds(base, CHUNK)],
                                         dst_v.at[b], sem_d)

        def scatter_d(b):
            return pltpu.make_async_copy(rows_v.at[b],
                                         agg_sh.at[dst_v.at[b]], sem_sc)

        def compute(b):
            @plsc.parallel_loop(0, CHUNK, unroll=4)
            def row_body(i):
                for j in range(D // (2 * LANES)):
                    sl_lo = pl.ds(2 * j * LANES, LANES)
                    sl_hi = pl.ds((2 * j + 1) * LANES, LANES)
                    u = ea_v[b, i, pl.ds(j * LANES, LANES)]
                    e_lo = jax.lax.bitcast_convert_type(u << 16, jnp.float32)
                    e_hi = jax.lax.bitcast_convert_type(
                        u & jnp.uint32(0xFFFF0000), jnp.float32)
                    rows_v[b, i, sl_lo] = jnp.maximum(
                        rows_v[b, i, sl_lo] + e_lo, 0.0)
                    rows_v[b, i, sl_hi] = jnp.maximum(
                        rows_v[b, i, sl_hi] + e_hi, 0.0)

        @plsc.parallel_loop(0, CHUNK, unroll=4)
        def zero_body(i):
            for j in range(D // LANES):
                rows_v[0, i, pl.ds(j * LANES, LANES)] = jnp.zeros(
                    (LANES,), jnp.float32)

        for zb in range(ROWS_PER_TILE // CHUNK):
            r0 = s * ROWS_PER_TILE + zb * CHUNK
            pltpu.sync_copy(rows_v.at[0], agg_sh.at[pl.ds(r0, CHUNK)])
        plsc.subcore_barrier()

        src_d(0, 0).start()
        src_d(1, 1).start()
        src_d(0, 0).wait()
        gather_d(0).start()
        ea_d(0, 0).start()
        dst_d(0, 0).start()

        def body(ci, b, prefetch=True):
            o = 1 - b

            @pl.when(ci >= 1)
            def _():
                scatter_d(o).wait()

            if prefetch:
                @pl.when(ci + 1 < NCHUNK)
                def _():
                    ea_d(ci + 1, o).start()
                    dst_d(ci + 1, o).start()
                    src_d(ci + 1, o).wait()
                    gather_d(o).start()

            gather_d(b).wait()

            if prefetch:
                @pl.when(ci + 2 < NCHUNK)
                def _():
                    src_d(ci + 2, b).start()

            ea_d(ci, b).wait()
            dst_d(ci, b).wait()
            compute(b)
            scatter_d(b).start(add=True)

        def outer_body(g, carry):
            for b in range(2):
                body(2 * g + b, b)
            return carry

        lax.fori_loop(0, NCHUNK // 2, outer_body, 0)
        body(NCHUNK - 1, 0, prefetch=False)
        scatter_d(0).wait()
        plsc.subcore_barrier()

        for b in range(ROWS_PER_TILE // CHUNK):
            r0 = s * ROWS_PER_TILE + b * CHUNK
            pltpu.sync_copy(agg_sh.at[pl.ds(r0, CHUNK)], rows_v.at[0])
            pltpu.sync_copy(rows_v.at[0], out_hbm.at[c, pl.ds(r0, CHUNK)])

    return stage_b


_stage_b = _stage_b_build()


def _stage_c_body(agg_ref, x_ref, wa_ref, ws_ref, bo_ref, a_ref, g_ref,
                  be_ref, out_ref):
    agg = (agg_ref[0] + agg_ref[1])[:N]
    h = jnp.dot(agg, wa_ref[...], preferred_element_type=jnp.float32)
    h = h + jnp.dot(x_ref[...], ws_ref[...], preferred_element_type=jnp.float32)
    h = h + bo_ref[...]
    slope = a_ref[0, 0]
    h = jnp.where(h >= 0, h, slope * h)
    mean = jnp.mean(h, axis=0, keepdims=True)
    var = jnp.mean((h - mean) ** 2, axis=0, keepdims=True)
    out_ref[...] = (h - mean) / jnp.sqrt(var + 1e-5) * g_ref[...] + be_ref[...]


_stage_c = pl.pallas_call(
    _stage_c_body,
    out_shape=jax.ShapeDtypeStruct((N, D), jnp.float32),
)


def kernel(x, edge_index, edge_attr, W_msg, W_edge, b_msg, W_agg, W_self,
           b_out, a, gamma, beta):
    src = edge_index[0]
    dst = edge_index[1]
    pi = jnp.asarray(_PI, dtype=jnp.int32)
    xm, ea = _stage_a(x, edge_attr, W_edge=None) if False else _stage_a(
        x, edge_attr, W_msg, W_edge[:, pi], b_msg[pi].reshape(1, D))
    zeros = jnp.zeros((N_PAD, D), dtype=jnp.float32)
    agg = _stage_b(xm, ea.reshape(E, D // 32, 32), zeros, src, dst)
    out = _stage_c(agg, x, W_agg, W_self, b_out.reshape(1, D),
                   a.reshape(1, 1), gamma.reshape(1, D), beta.reshape(1, D))
    return out
